# baseline (speedup 1.0000x reference)
import jax
import jax.numpy as jnp
from jax.experimental import pallas as pl
from jax.experimental.pallas import tpu as pltpu

_N = 8192
_B = 256
_NB = _N // _B
_NCHUNK = 16


def _body(x_hbm, o_hbm, blk_vmem, copy_sem, blk_sem):
    rows_per_chunk = _N // _NCHUNK
    bulk = [
        pltpu.make_async_copy(
            x_hbm.at[pl.ds(c * rows_per_chunk, rows_per_chunk), :],
            o_hbm.at[pl.ds(c * rows_per_chunk, rows_per_chunk), :],
            copy_sem,
        )
        for c in range(_NCHUNK)
    ]
    for cp in bulk:
        cp.start()
    gathers = [
        pltpu.make_async_copy(
            x_hbm.at[pl.ds(b * _B, _B), pl.ds(b * _B, _B)],
            blk_vmem.at[b],
            blk_sem,
        )
        for b in range(_NB)
    ]
    for cp in gathers:
        cp.start()
    for cp in gathers:
        cp.wait()
    blks = blk_vmem[...]
    r = jax.lax.broadcasted_iota(jnp.int32, (_NB, _B, _B), 1)
    c = jax.lax.broadcasted_iota(jnp.int32, (_NB, _B, _B), 2)
    blk_vmem[...] = jnp.where(r == c, jnp.tanh(blks), blks)
    for cp in bulk:
        cp.wait()
    scatters = [
        pltpu.make_async_copy(
            blk_vmem.at[b],
            o_hbm.at[pl.ds(b * _B, _B), pl.ds(b * _B, _B)],
            blk_sem,
        )
        for b in range(_NB)
    ]
    for cp in scatters:
        cp.start()
    for cp in scatters:
        cp.wait()


def kernel(x):
    n = x.shape[0]
    return pl.pallas_call(
        _body,
        in_specs=[pl.BlockSpec(memory_space=pltpu.MemorySpace.HBM)],
        out_specs=pl.BlockSpec(memory_space=pltpu.MemorySpace.HBM),
        out_shape=jax.ShapeDtypeStruct((n, n), x.dtype),
        scratch_shapes=[
            pltpu.VMEM((_NB, _B, _B), jnp.float32),
            pltpu.SemaphoreType.DMA,
            pltpu.SemaphoreType.DMA,
        ],
    )(x)


# speedup vs baseline: 48.6190x; 48.6190x over previous
import jax
import jax.numpy as jnp
from jax.experimental import pallas as pl

_N = 8192
_BR = 128


def _body(x_ref, o_ref):
    i = pl.program_id(0)
    o_ref[...] = x_ref[...]
    c0 = i * _BR
    sub = x_ref[:, pl.ds(c0, _BR)]
    rows = jax.lax.broadcasted_iota(jnp.int32, (_BR, _BR), 0)
    cols = jax.lax.broadcasted_iota(jnp.int32, (_BR, _BR), 1)
    o_ref[:, pl.ds(c0, _BR)] = jnp.where(rows == cols, jnp.tanh(sub), sub)


def kernel(x):
    n = x.shape[0]
    return pl.pallas_call(
        _body,
        grid=(n // _BR,),
        in_specs=[pl.BlockSpec((_BR, n), lambda i: (i, 0))],
        out_specs=pl.BlockSpec((_BR, n), lambda i: (i, 0)),
        out_shape=jax.ShapeDtypeStruct((n, n), x.dtype),
    )(x)
